# baseline (speedup 1.0000x reference)
import functools

import jax
import jax.numpy as jnp
from jax import lax
from jax.experimental import pallas as pl
from jax.experimental.pallas import tpu as pltpu
from jax.experimental.pallas import tpu_sc as plsc

_N = 10000
_D = 128
_E = 320000
_RB = 400
_CH = 128
_NSUB = 16
_CPT = 160
_G = 32
_EPAD = _NSUB * _CPT * _CH
_RPT = 624
_TAIL0 = _NSUB * _RPT
_TAILN = _N - _TAIL0


def _tc_body(mean_ref, var_ref, wm_ref, bm_ref, wv_ref, bv_ref, out_ref):
    dn = (((1,), (1,)), ((), ()))
    ml = lax.dot_general(mean_ref[...], wm_ref[...], dn,
                         preferred_element_type=jnp.float32) + bm_ref[...]
    vl = lax.dot_general(var_ref[...], wv_ref[...], dn,
                         preferred_element_type=jnp.float32) + bv_ref[...]
    me = jnp.where(ml > 0, ml, jnp.exp(ml) - 1.0)
    vr = jnp.maximum(vl, 0.0)
    att = jnp.exp(-vr)
    out_ref[0] = me * att
    out_ref[1] = vr * att * att


def _tc_transform(mean, var, w_mean, b_mean, w_var, b_var):
    nb = _N // _RB
    return pl.pallas_call(
        _tc_body,
        grid=(nb,),
        in_specs=[
            pl.BlockSpec((_RB, _D), lambda b: (b, 0)),
            pl.BlockSpec((_RB, _D), lambda b: (b, 0)),
            pl.BlockSpec((_D, _D), lambda b: (0, 0)),
            pl.BlockSpec((1, _D), lambda b: (0, 0)),
            pl.BlockSpec((_D, _D), lambda b: (0, 0)),
            pl.BlockSpec((1, _D), lambda b: (0, 0)),
        ],
        out_specs=pl.BlockSpec((2, _RB, _D), lambda b: (0, b, 0)),
        out_shape=jax.ShapeDtypeStruct((2, _N, _D), jnp.float32),
    )(mean, var, w_mean, b_mean.reshape(1, _D), w_var, b_var.reshape(1, _D))


def _sc_body(x_hbm, cols_hbm, rows_hbm, adj_hbm, m_out, v_out,
             col_a, row_a, w_a, rows0, rows1, acc,
             isem, gsem0, gsem1, ssem0, ssem1):
    cid = lax.axis_index("c")
    sid = lax.axis_index("s")

    def _zrow(i, carry):
        for b in range(8):
            rows0[i, pl.ds(b * 16, 16)] = jnp.zeros((16,), jnp.float32)
        return carry
    lax.fori_loop(0, _CH, _zrow, 0)
    r0 = sid * _RPT
    for k in range(4):
        pltpu.sync_copy(rows0, acc.at[pl.ds(r0 + k * _CH, _CH)])
    pltpu.sync_copy(rows0.at[pl.ds(0, _RPT - 4 * _CH)],
                    acc.at[pl.ds(r0 + 4 * _CH, _RPT - 4 * _CH)])

    @pl.when(sid == _NSUB - 1)
    def _():
        pltpu.sync_copy(rows0.at[pl.ds(0, _TAILN)],
                        acc.at[pl.ds(_TAIL0, _TAILN)])
    plsc.subcore_barrier()

    def _gather(c, buf, sem):
        return pltpu.async_copy(x_hbm.at[col_a.at[c]], buf, sem)

    def _scatter(c, buf, sem):
        return pltpu.async_copy(buf, acc.at[row_a.at[c]], sem, add=True)

    def _wait_gather(buf, sem):
        pltpu.make_async_copy(x_hbm.at[col_a.at[0]], buf, sem).wait()

    def _wait_scatter(buf, sem):
        pltpu.make_async_copy(buf, acc.at[row_a.at[0]], sem).wait()

    def _scale(c, buf):
        def _grp(g, carry):
            wv = w_a[c, pl.ds(g * 16, 16)]
            for k in range(16):
                wbc = jnp.broadcast_to(wv[k], (16,))
                e = g * 16 + k
                for b in range(8):
                    sl = pl.ds(b * 16, 16)
                    buf[e, sl] = buf[e, sl] * wbc
            return carry
        lax.fori_loop(0, _CH // 16, _grp, 0)

    def _batch(t, carry):
        c0 = sid * _CPT + t * _G
        dcol = pltpu.async_copy(cols_hbm.at[cid, pl.ds(c0, _G)], col_a, isem)
        drow = pltpu.async_copy(rows_hbm.at[pl.ds(c0, _G)], row_a, isem)
        dadj = pltpu.async_copy(adj_hbm.at[cid, pl.ds(c0, _G)], w_a, isem)
        dcol.wait()
        drow.wait()
        dadj.wait()

        _gather(0, rows0, gsem0)
        _gather(1, rows1, gsem1)
        _wait_gather(rows0, gsem0)
        _scale(0, rows0)
        _scatter(0, rows0, ssem0)

        def _pair(k, c2):
            ca = 2 * k + 1
            cb = 2 * k + 2
            _wait_gather(rows1, gsem1)
            _scale(ca, rows1)
            _wait_scatter(rows0, ssem0)
            _gather(cb, rows0, gsem0)
            _scatter(ca, rows1, ssem1)
            _wait_gather(rows0, gsem0)
            _scale(cb, rows0)
            _wait_scatter(rows1, ssem1)
            _gather(cb + 1, rows1, gsem1)
            _scatter(cb, rows0, ssem0)
            return c2
        lax.fori_loop(0, (_G - 2) // 2, _pair, 0)

        _wait_gather(rows1, gsem1)
        _scale(_G - 1, rows1)
        _scatter(_G - 1, rows1, ssem1)
        _wait_scatter(rows0, ssem0)
        _wait_scatter(rows1, ssem1)
        return carry
    lax.fori_loop(0, _CPT // _G, _batch, 0)
    plsc.subcore_barrier()

    @pl.when(cid == 0)
    def _():
        pltpu.sync_copy(acc.at[pl.ds(r0, _RPT)], m_out.at[pl.ds(r0, _RPT)])

        @pl.when(sid == _NSUB - 1)
        def _():
            pltpu.sync_copy(acc.at[pl.ds(_TAIL0, _TAILN)],
                            m_out.at[pl.ds(_TAIL0, _TAILN)])

    @pl.when(cid == 1)
    def _():
        pltpu.sync_copy(acc.at[pl.ds(r0, _RPT)], v_out.at[pl.ds(r0, _RPT)])

        @pl.when(sid == _NSUB - 1)
        def _():
            pltpu.sync_copy(acc.at[pl.ds(_TAIL0, _TAILN)],
                            v_out.at[pl.ds(_TAIL0, _TAILN)])


@functools.cache
def _sc_aggregate():
    return functools.partial(
        pl.kernel,
        out_type=[jax.ShapeDtypeStruct((_N, _D), jnp.float32),
                  jax.ShapeDtypeStruct((_N, _D), jnp.float32)],
        mesh=plsc.VectorSubcoreMesh(core_axis_name="c", subcore_axis_name="s",
                                    num_cores=2, num_subcores=_NSUB),
        scratch_types=[
            pltpu.VMEM((_G, _CH), jnp.int32),
            pltpu.VMEM((_G, _CH), jnp.int32),
            pltpu.VMEM((_G, _CH), jnp.float32),
            pltpu.VMEM((_CH, _D), jnp.float32),
            pltpu.VMEM((_CH, _D), jnp.float32),
            pltpu.VMEM_SHARED((_N, _D), jnp.float32),
            pltpu.SemaphoreType.DMA,
            pltpu.SemaphoreType.DMA,
            pltpu.SemaphoreType.DMA,
            pltpu.SemaphoreType.DMA,
            pltpu.SemaphoreType.DMA,
        ],
    )(_sc_body)


def kernel(mean, var, edge_index, adj0_values, adj1_values,
           W_mean, b_mean, W_var, b_var):
    x_all = _tc_transform(mean, var, W_mean, b_mean, W_var, b_var)
    x_all = x_all.reshape(2 * _N, _D)
    pad = _EPAD - _E
    col = jnp.pad(edge_index[1], (0, pad))
    row = jnp.pad(edge_index[0], (0, pad))
    cols_all = jnp.stack([col, col + _N]).reshape(2, _EPAD // _CH, _CH)
    rows_all = row.reshape(_EPAD // _CH, _CH)
    adj_all = jnp.stack([jnp.pad(adj0_values, (0, pad)),
                         jnp.pad(adj1_values, (0, pad))])
    adj_all = adj_all.reshape(2, _EPAD // _CH, _CH)
    m_out, v_out = _sc_aggregate()(x_all, cols_all, rows_all, adj_all)
    return (m_out, v_out)


# speedup vs baseline: 1.0738x; 1.0738x over previous
import functools

import jax
import jax.numpy as jnp
from jax import lax
from jax.experimental import pallas as pl
from jax.experimental.pallas import tpu as pltpu
from jax.experimental.pallas import tpu_sc as plsc

_N = 10000
_D = 128
_E = 320000
_RB = 400
_CH = 64
_NSUB = 16
_CPT = 320
_EPAD = _NSUB * _CPT * _CH
_EHOST = _EPAD + 4 * _CH
_RPT = 624
_TAIL0 = _NSUB * _RPT
_TAILN = _N - _TAIL0
_UNROLL = 8


def _tc_body(mean_ref, var_ref, wm_ref, bm_ref, wv_ref, bv_ref, out_ref):
    dn = (((1,), (1,)), ((), ()))
    ml = lax.dot_general(mean_ref[...], wm_ref[...], dn,
                         preferred_element_type=jnp.float32) + bm_ref[...]
    vl = lax.dot_general(var_ref[...], wv_ref[...], dn,
                         preferred_element_type=jnp.float32) + bv_ref[...]
    me = jnp.where(ml > 0, ml, jnp.exp(ml) - 1.0)
    vr = jnp.maximum(vl, 0.0)
    att = jnp.exp(-vr)
    out_ref[0] = me * att
    out_ref[1] = vr * att * att


def _tc_transform(mean, var, w_mean, b_mean, w_var, b_var):
    nb = _N // _RB
    return pl.pallas_call(
        _tc_body,
        grid=(nb,),
        in_specs=[
            pl.BlockSpec((_RB, _D), lambda b: (b, 0)),
            pl.BlockSpec((_RB, _D), lambda b: (b, 0)),
            pl.BlockSpec((_D, _D), lambda b: (0, 0)),
            pl.BlockSpec((1, _D), lambda b: (0, 0)),
            pl.BlockSpec((_D, _D), lambda b: (0, 0)),
            pl.BlockSpec((1, _D), lambda b: (0, 0)),
        ],
        out_specs=pl.BlockSpec((2, _RB, _D), lambda b: (0, b, 0)),
        out_shape=jax.ShapeDtypeStruct((2, _N, _D), jnp.float32),
    )(mean, var, w_mean, b_mean.reshape(1, _D), w_var, b_var.reshape(1, _D))


def _sc_body(x_hbm, cols_hbm, rows_hbm, adj_hbm, m_out, v_out,
             col_r, row_r, w_r, buf0, buf1, buf2, buf3, acc, sems):
    cid = lax.axis_index("c")
    sid = lax.axis_index("s")
    bufs = (buf0, buf1, buf2, buf3)
    e00 = sid * _CPT * _CH

    def _stage_idx(c, slot):
        e0 = e00 + c * _CH
        pltpu.async_copy(cols_hbm.at[cid, pl.ds(e0, _CH)],
                         col_r.at[slot], sems[8 + slot])
        pltpu.async_copy(rows_hbm.at[pl.ds(e0, _CH)],
                         row_r.at[slot], sems[8 + slot])
        pltpu.async_copy(adj_hbm.at[cid, pl.ds(e0, _CH)],
                         w_r.at[slot], sems[8 + slot])

    def _wait_idx(slot):
        pltpu.make_async_copy(cols_hbm.at[cid, pl.ds(0, _CH)],
                              col_r.at[slot], sems[8 + slot]).wait()
        pltpu.make_async_copy(rows_hbm.at[pl.ds(0, _CH)],
                              row_r.at[slot], sems[8 + slot]).wait()
        pltpu.make_async_copy(adj_hbm.at[cid, pl.ds(0, _CH)],
                              w_r.at[slot], sems[8 + slot]).wait()

    def _gather(slot, b):
        pltpu.async_copy(x_hbm.at[col_r.at[slot]], bufs[b], sems[b])

    def _wait_gather(b):
        pltpu.make_async_copy(x_hbm.at[col_r.at[0]], bufs[b], sems[b]).wait()

    def _scatter(slot, b):
        pltpu.async_copy(bufs[b], acc.at[row_r.at[slot]], sems[4 + b],
                         add=True)

    def _wait_scatter(b):
        pltpu.make_async_copy(bufs[b], acc.at[row_r.at[0]],
                              sems[4 + b]).wait()

    def _scale(slot, b):
        buf = bufs[b]

        def _grp(g, carry):
            wv = w_r[slot, pl.ds(g * 16, 16)]
            for k in range(16):
                wbc = jnp.broadcast_to(wv[k], (16,))
                e = g * 16 + k
                for fb in range(8):
                    sl = pl.ds(fb * 16, 16)
                    buf[e, sl] = buf[e, sl] * wbc
            return carry
        lax.fori_loop(0, _CH // 16, _grp, 0)

    for c in range(4):
        _stage_idx(c, c)

    def _zrow(i, carry):
        for b in range(8):
            buf0[i, pl.ds(b * 16, 16)] = jnp.zeros((16,), jnp.float32)
        return carry
    lax.fori_loop(0, _CH, _zrow, 0)
    r0 = sid * _RPT
    for k in range(9):
        pltpu.sync_copy(buf0, acc.at[pl.ds(r0 + k * _CH, _CH)])
    pltpu.sync_copy(buf0.at[pl.ds(0, _RPT - 9 * _CH)],
                    acc.at[pl.ds(r0 + 9 * _CH, _RPT - 9 * _CH)])

    @pl.when(sid == _NSUB - 1)
    def _():
        pltpu.sync_copy(buf0.at[pl.ds(0, _TAILN)],
                        acc.at[pl.ds(_TAIL0, _TAILN)])
    plsc.subcore_barrier()

    pltpu.async_copy(acc.at[pl.ds(0, _CH)], buf2, sems[4 + 2])
    pltpu.async_copy(acc.at[pl.ds(0, _CH)], buf3, sems[4 + 3])

    _wait_idx(0)
    _wait_idx(1)
    _gather(0, 0)
    _gather(1, 1)

    def _body(it, carry):
        c_base = it * _UNROLL
        for u in range(_UNROLL):
            b = u % 4
            s = u % 8
            _wait_gather(b)
            _scale(s, b)
            _wait_scatter((u + 2) % 4)
            _wait_idx((u + 2) % 8)
            _gather((u + 2) % 8, (u + 2) % 4)
            _stage_idx(c_base + u + 4, (u + 4) % 8)
            _scatter(s, b)
        return carry
    lax.fori_loop(0, _CPT // _UNROLL, _body, 0)

    _wait_gather(0)
    _wait_gather(1)
    _wait_idx(2)
    _wait_idx(3)
    _wait_scatter(2)
    _wait_scatter(3)
    plsc.subcore_barrier()

    @pl.when(cid == 0)
    def _():
        pltpu.sync_copy(acc.at[pl.ds(r0, _RPT)], m_out.at[pl.ds(r0, _RPT)])

        @pl.when(sid == _NSUB - 1)
        def _():
            pltpu.sync_copy(acc.at[pl.ds(_TAIL0, _TAILN)],
                            m_out.at[pl.ds(_TAIL0, _TAILN)])

    @pl.when(cid == 1)
    def _():
        pltpu.sync_copy(acc.at[pl.ds(r0, _RPT)], v_out.at[pl.ds(r0, _RPT)])

        @pl.when(sid == _NSUB - 1)
        def _():
            pltpu.sync_copy(acc.at[pl.ds(_TAIL0, _TAILN)],
                            v_out.at[pl.ds(_TAIL0, _TAILN)])


@functools.cache
def _sc_aggregate():
    return functools.partial(
        pl.kernel,
        out_type=[jax.ShapeDtypeStruct((_N, _D), jnp.float32),
                  jax.ShapeDtypeStruct((_N, _D), jnp.float32)],
        mesh=plsc.VectorSubcoreMesh(core_axis_name="c", subcore_axis_name="s",
                                    num_cores=2, num_subcores=_NSUB),
        scratch_types=[
            pltpu.VMEM((8, _CH), jnp.int32),
            pltpu.VMEM((8, _CH), jnp.int32),
            pltpu.VMEM((8, _CH), jnp.float32),
            pltpu.VMEM((_CH, _D), jnp.float32),
            pltpu.VMEM((_CH, _D), jnp.float32),
            pltpu.VMEM((_CH, _D), jnp.float32),
            pltpu.VMEM((_CH, _D), jnp.float32),
            pltpu.VMEM_SHARED((_N, _D), jnp.float32),
            [pltpu.SemaphoreType.DMA] * 16,
        ],
    )(_sc_body)


def kernel(mean, var, edge_index, adj0_values, adj1_values,
           W_mean, b_mean, W_var, b_var):
    x_all = _tc_transform(mean, var, W_mean, b_mean, W_var, b_var)
    x_all = x_all.reshape(2 * _N, _D)
    pad = _EHOST - _E
    col = jnp.pad(edge_index[1], (0, pad))
    row = jnp.pad(edge_index[0], (0, pad))
    cols_all = jnp.stack([col, col + _N])
    adj_all = jnp.stack([jnp.pad(adj0_values, (0, pad)),
                         jnp.pad(adj1_values, (0, pad))])
    m_out, v_out = _sc_aggregate()(x_all, cols_all, row, adj_all)
    return (m_out, v_out)


# speedup vs baseline: 1.8964x; 1.7661x over previous
import functools

import jax
import jax.numpy as jnp
from jax import lax
from jax.experimental import pallas as pl
from jax.experimental.pallas import tpu as pltpu
from jax.experimental.pallas import tpu_sc as plsc

_N = 10000
_D = 128
_E = 320000
_RB = 400
_CH = 96
_NSUB = 16
_CPT = 210
_EPAD = _NSUB * _CPT * _CH
_EHOST = _EPAD + 3 * _CH
_RPT = 624
_TAIL0 = _NSUB * _RPT
_TAILN = _N - _TAIL0
_UNROLL = 15


def _tc_body(mean_ref, var_ref, wm_ref, bm_ref, wv_ref, bv_ref, out_ref):
    dn = (((1,), (1,)), ((), ()))
    ml = lax.dot_general(mean_ref[...], wm_ref[...], dn,
                         preferred_element_type=jnp.float32) + bm_ref[...]
    vl = lax.dot_general(var_ref[...], wv_ref[...], dn,
                         preferred_element_type=jnp.float32) + bv_ref[...]
    me = jnp.where(ml > 0, ml, jnp.exp(ml) - 1.0)
    vr = jnp.maximum(vl, 0.0)
    att = jnp.exp(-vr)
    out_ref[0] = me * att
    out_ref[1] = vr * att * att


def _tc_transform(mean, var, w_mean, b_mean, w_var, b_var):
    nb = _N // _RB
    return pl.pallas_call(
        _tc_body,
        grid=(nb,),
        in_specs=[
            pl.BlockSpec((_RB, _D), lambda b: (b, 0)),
            pl.BlockSpec((_RB, _D), lambda b: (b, 0)),
            pl.BlockSpec((_D, _D), lambda b: (0, 0)),
            pl.BlockSpec((1, _D), lambda b: (0, 0)),
            pl.BlockSpec((_D, _D), lambda b: (0, 0)),
            pl.BlockSpec((1, _D), lambda b: (0, 0)),
        ],
        out_specs=pl.BlockSpec((2, _RB, _D), lambda b: (0, b, 0)),
        out_shape=jax.ShapeDtypeStruct((2, _N, _D), jnp.float32),
    )(mean, var, w_mean, b_mean.reshape(1, _D), w_var, b_var.reshape(1, _D))


def _sc_body(x_hbm, cols_hbm, rows_hbm, adj_hbm, m_out, v_out,
             col_r, row_r, w_r, buf0, buf1, buf2, acc, sems):
    cid = lax.axis_index("c")
    sid = lax.axis_index("s")
    bufs = (buf0, buf1, buf2)
    e00 = sid * _CPT * _CH

    def _stage_idx(c, slot):
        e0 = cid * _EHOST + e00 + c * _CH
        pltpu.async_copy(cols_hbm.at[pl.ds(e0, _CH)],
                         col_r.at[slot], sems[6 + slot])
        pltpu.async_copy(rows_hbm.at[pl.ds(e00 + c * _CH, _CH)],
                         row_r.at[slot], sems[6 + slot])
        pltpu.async_copy(adj_hbm.at[pl.ds(e0, _CH)],
                         w_r.at[slot], sems[6 + slot])

    def _wait_idx(slot):
        pltpu.make_async_copy(cols_hbm.at[pl.ds(0, _CH)],
                              col_r.at[slot], sems[6 + slot]).wait()
        pltpu.make_async_copy(rows_hbm.at[pl.ds(0, _CH)],
                              row_r.at[slot], sems[6 + slot]).wait()
        pltpu.make_async_copy(adj_hbm.at[pl.ds(0, _CH)],
                              w_r.at[slot], sems[6 + slot]).wait()

    def _gather(slot, b):
        pltpu.async_copy(x_hbm.at[col_r.at[slot]], bufs[b], sems[b])

    def _wait_gather(b):
        pltpu.make_async_copy(x_hbm.at[col_r.at[0]], bufs[b], sems[b]).wait()

    def _scatter(slot, b):
        pltpu.async_copy(bufs[b], acc.at[row_r.at[slot]], sems[3 + b],
                         add=True)

    def _wait_scatter(b):
        pltpu.make_async_copy(bufs[b], acc.at[row_r.at[0]],
                              sems[3 + b]).wait()

    def _scale(slot, b):
        buf = bufs[b]

        def _grp(g, carry):
            wv = w_r[slot, pl.ds(g * 16, 16)]
            for k in range(16):
                wbc = jnp.broadcast_to(wv[k], (16,))
                e = g * 16 + k
                for fb in range(8):
                    sl = pl.ds(fb * 16, 16)
                    buf[e, sl] = buf[e, sl] * wbc
            return carry
        lax.fori_loop(0, _CH // 16, _grp, 0)

    for c in range(3):
        _stage_idx(c, c)

    def _zrow(i, carry):
        for b in range(8):
            buf0[i, pl.ds(b * 16, 16)] = jnp.zeros((16,), jnp.float32)
        return carry
    lax.fori_loop(0, _CH, _zrow, 0)
    r0 = sid * _RPT
    for k in range(6):
        pltpu.sync_copy(buf0, acc.at[pl.ds(r0 + k * _CH, _CH)])
    pltpu.sync_copy(buf0.at[pl.ds(0, _RPT - 6 * _CH)],
                    acc.at[pl.ds(r0 + 6 * _CH, _RPT - 6 * _CH)])

    @pl.when(sid == _NSUB - 1)
    def _():
        pltpu.sync_copy(buf0.at[pl.ds(0, _TAILN)],
                        acc.at[pl.ds(_TAIL0, _TAILN)])
    plsc.subcore_barrier()

    pltpu.async_copy(acc.at[pl.ds(0, _CH)], buf2, sems[3 + 2])

    _wait_idx(0)
    _wait_idx(1)
    _gather(0, 0)
    _gather(1, 1)

    def _body(it, carry):
        c_base = it * _UNROLL
        for u in range(_UNROLL):
            b = u % 3
            s = u % 5
            _wait_gather(b)
            _scale(s, b)
            _wait_scatter((u + 2) % 3)
            _wait_idx((u + 2) % 5)
            _gather((u + 2) % 5, (u + 2) % 3)
            _stage_idx(c_base + u + 3, (u + 3) % 5)
            _scatter(s, b)
        return carry
    lax.fori_loop(0, _CPT // _UNROLL, _body, 0)

    _wait_gather(0)
    _wait_gather(1)
    _wait_idx(212 % 5)
    _wait_scatter(209 % 3)
    plsc.subcore_barrier()

    @pl.when(cid == 0)
    def _():
        pltpu.sync_copy(acc.at[pl.ds(r0, _RPT)], m_out.at[pl.ds(r0, _RPT)])

        @pl.when(sid == _NSUB - 1)
        def _():
            pltpu.sync_copy(acc.at[pl.ds(_TAIL0, _TAILN)],
                            m_out.at[pl.ds(_TAIL0, _TAILN)])

    @pl.when(cid == 1)
    def _():
        pltpu.sync_copy(acc.at[pl.ds(r0, _RPT)], v_out.at[pl.ds(r0, _RPT)])

        @pl.when(sid == _NSUB - 1)
        def _():
            pltpu.sync_copy(acc.at[pl.ds(_TAIL0, _TAILN)],
                            v_out.at[pl.ds(_TAIL0, _TAILN)])


@functools.cache
def _sc_aggregate():
    return functools.partial(
        pl.kernel,
        out_type=[jax.ShapeDtypeStruct((_N, _D), jnp.float32),
                  jax.ShapeDtypeStruct((_N, _D), jnp.float32)],
        mesh=plsc.VectorSubcoreMesh(core_axis_name="c", subcore_axis_name="s",
                                    num_cores=2, num_subcores=_NSUB),
        scratch_types=[
            pltpu.VMEM((5, _CH), jnp.int32),
            pltpu.VMEM((5, _CH), jnp.int32),
            pltpu.VMEM((5, _CH), jnp.float32),
            pltpu.VMEM((_CH, _D), jnp.float32),
            pltpu.VMEM((_CH, _D), jnp.float32),
            pltpu.VMEM((_CH, _D), jnp.float32),
            pltpu.VMEM_SHARED((_N, _D), jnp.float32),
            [pltpu.SemaphoreType.DMA] * 11,
        ],
    )(_sc_body)


def kernel(mean, var, edge_index, adj0_values, adj1_values,
           W_mean, b_mean, W_var, b_var):
    x_all = _tc_transform(mean, var, W_mean, b_mean, W_var, b_var)
    x_all = x_all.reshape(2 * _N, _D)
    pad = _EHOST - _E
    col = jnp.pad(edge_index[1], (0, pad))
    row = jnp.pad(edge_index[0], (0, pad))
    cols_all = jnp.concatenate([col, col + _N])
    adj_all = jnp.concatenate([jnp.pad(adj0_values, (0, pad)),
                               jnp.pad(adj1_values, (0, pad))])
    m_out, v_out = _sc_aggregate()(x_all, cols_all, row, adj_all)
    return (m_out, v_out)


# speedup vs baseline: 1.9017x; 1.0028x over previous
import functools

import jax
import jax.numpy as jnp
from jax import lax
from jax.experimental import pallas as pl
from jax.experimental.pallas import tpu as pltpu
from jax.experimental.pallas import tpu_sc as plsc

_N = 10000
_D = 128
_E = 320000
_RB = 400
_CH = 112
_NSUB = 16
_CPT = 180
_EPAD = _NSUB * _CPT * _CH
_EHOST = _EPAD + 3 * _CH
_RPT = 624
_TAIL0 = _NSUB * _RPT
_TAILN = _N - _TAIL0
_UNROLL = 15


def _tc_body(mean_ref, var_ref, wm_ref, bm_ref, wv_ref, bv_ref, out_ref):
    dn = (((1,), (1,)), ((), ()))
    ml = lax.dot_general(mean_ref[...], wm_ref[...], dn,
                         preferred_element_type=jnp.float32) + bm_ref[...]
    vl = lax.dot_general(var_ref[...], wv_ref[...], dn,
                         preferred_element_type=jnp.float32) + bv_ref[...]
    me = jnp.where(ml > 0, ml, jnp.exp(ml) - 1.0)
    vr = jnp.maximum(vl, 0.0)
    att = jnp.exp(-vr)
    out_ref[0] = me * att
    out_ref[1] = vr * att * att


def _tc_transform(mean, var, w_mean, b_mean, w_var, b_var):
    nb = _N // _RB
    return pl.pallas_call(
        _tc_body,
        grid=(nb,),
        in_specs=[
            pl.BlockSpec((_RB, _D), lambda b: (b, 0)),
            pl.BlockSpec((_RB, _D), lambda b: (b, 0)),
            pl.BlockSpec((_D, _D), lambda b: (0, 0)),
            pl.BlockSpec((1, _D), lambda b: (0, 0)),
            pl.BlockSpec((_D, _D), lambda b: (0, 0)),
            pl.BlockSpec((1, _D), lambda b: (0, 0)),
        ],
        out_specs=pl.BlockSpec((2, _RB, _D), lambda b: (0, b, 0)),
        out_shape=jax.ShapeDtypeStruct((2, _N, _D), jnp.float32),
    )(mean, var, w_mean, b_mean.reshape(1, _D), w_var, b_var.reshape(1, _D))


def _sc_body(x_hbm, cols_hbm, rows_hbm, adj_hbm, m_out, v_out,
             col_r, row_r, w_r, buf0, buf1, buf2, acc, sems):
    cid = lax.axis_index("c")
    sid = lax.axis_index("s")
    bufs = (buf0, buf1, buf2)
    e00 = sid * _CPT * _CH

    def _stage_idx(c, slot):
        e0 = cid * _EHOST + e00 + c * _CH
        pltpu.async_copy(cols_hbm.at[pl.ds(e0, _CH)],
                         col_r.at[slot], sems[6 + slot])
        pltpu.async_copy(rows_hbm.at[pl.ds(e00 + c * _CH, _CH)],
                         row_r.at[slot], sems[6 + slot])
        pltpu.async_copy(adj_hbm.at[pl.ds(e0, _CH)],
                         w_r.at[slot], sems[6 + slot])

    def _wait_idx(slot):
        pltpu.make_async_copy(cols_hbm.at[pl.ds(0, _CH)],
                              col_r.at[slot], sems[6 + slot]).wait()
        pltpu.make_async_copy(rows_hbm.at[pl.ds(0, _CH)],
                              row_r.at[slot], sems[6 + slot]).wait()
        pltpu.make_async_copy(adj_hbm.at[pl.ds(0, _CH)],
                              w_r.at[slot], sems[6 + slot]).wait()

    def _gather(slot, b):
        pltpu.async_copy(x_hbm.at[col_r.at[slot]], bufs[b], sems[b])

    def _wait_gather(b):
        pltpu.make_async_copy(x_hbm.at[col_r.at[0]], bufs[b], sems[b]).wait()

    def _scatter(slot, b):
        pltpu.async_copy(bufs[b], acc.at[row_r.at[slot]], sems[3 + b],
                         add=True)

    def _wait_scatter(b):
        pltpu.make_async_copy(bufs[b], acc.at[row_r.at[0]],
                              sems[3 + b]).wait()

    def _scale(slot, b):
        buf = bufs[b]

        def _grp(g, carry):
            wv = w_r[slot, pl.ds(g * 16, 16)]
            for k in range(16):
                wbc = jnp.broadcast_to(wv[k], (16,))
                e = g * 16 + k
                for fb in range(8):
                    sl = pl.ds(fb * 16, 16)
                    buf[e, sl] = buf[e, sl] * wbc
            return carry
        lax.fori_loop(0, _CH // 16, _grp, 0)

    for c in range(3):
        _stage_idx(c, c)

    def _zrow(i, carry):
        for b in range(8):
            buf0[i, pl.ds(b * 16, 16)] = jnp.zeros((16,), jnp.float32)
        return carry
    lax.fori_loop(0, _CH, _zrow, 0)
    r0 = sid * _RPT
    for k in range(5):
        pltpu.sync_copy(buf0, acc.at[pl.ds(r0 + k * _CH, _CH)])
    pltpu.sync_copy(buf0.at[pl.ds(0, _RPT - 5 * _CH)],
                    acc.at[pl.ds(r0 + 5 * _CH, _RPT - 5 * _CH)])

    @pl.when(sid == _NSUB - 1)
    def _():
        pltpu.sync_copy(buf0.at[pl.ds(0, _TAILN)],
                        acc.at[pl.ds(_TAIL0, _TAILN)])
    plsc.subcore_barrier()

    pltpu.async_copy(acc.at[pl.ds(0, _CH)], buf2, sems[3 + 2])

    _wait_idx(0)
    _wait_idx(1)
    _gather(0, 0)
    _gather(1, 1)

    def _body(it, carry):
        c_base = it * _UNROLL
        for u in range(_UNROLL):
            b = u % 3
            s = u % 5
            _wait_gather(b)
            _scale(s, b)
            _wait_scatter((u + 2) % 3)
            _wait_idx((u + 2) % 5)
            _gather((u + 2) % 5, (u + 2) % 3)
            _stage_idx(c_base + u + 3, (u + 3) % 5)
            _scatter(s, b)
        return carry
    lax.fori_loop(0, _CPT // _UNROLL, _body, 0)

    _wait_gather(0)
    _wait_gather(1)
    _wait_idx(182 % 5)
    _wait_scatter(179 % 3)
    plsc.subcore_barrier()

    @pl.when(cid == 0)
    def _():
        pltpu.sync_copy(acc.at[pl.ds(r0, _RPT)], m_out.at[pl.ds(r0, _RPT)])

        @pl.when(sid == _NSUB - 1)
        def _():
            pltpu.sync_copy(acc.at[pl.ds(_TAIL0, _TAILN)],
                            m_out.at[pl.ds(_TAIL0, _TAILN)])

    @pl.when(cid == 1)
    def _():
        pltpu.sync_copy(acc.at[pl.ds(r0, _RPT)], v_out.at[pl.ds(r0, _RPT)])

        @pl.when(sid == _NSUB - 1)
        def _():
            pltpu.sync_copy(acc.at[pl.ds(_TAIL0, _TAILN)],
                            v_out.at[pl.ds(_TAIL0, _TAILN)])


@functools.cache
def _sc_aggregate():
    return functools.partial(
        pl.kernel,
        out_type=[jax.ShapeDtypeStruct((_N, _D), jnp.float32),
                  jax.ShapeDtypeStruct((_N, _D), jnp.float32)],
        mesh=plsc.VectorSubcoreMesh(core_axis_name="c", subcore_axis_name="s",
                                    num_cores=2, num_subcores=_NSUB),
        scratch_types=[
            pltpu.VMEM((5, _CH), jnp.int32),
            pltpu.VMEM((5, _CH), jnp.int32),
            pltpu.VMEM((5, _CH), jnp.float32),
            pltpu.VMEM((_CH, _D), jnp.float32),
            pltpu.VMEM((_CH, _D), jnp.float32),
            pltpu.VMEM((_CH, _D), jnp.float32),
            pltpu.VMEM_SHARED((_N, _D), jnp.float32),
            [pltpu.SemaphoreType.DMA] * 11,
        ],
    )(_sc_body)


def kernel(mean, var, edge_index, adj0_values, adj1_values,
           W_mean, b_mean, W_var, b_var):
    x_all = _tc_transform(mean, var, W_mean, b_mean, W_var, b_var)
    x_all = x_all.reshape(2 * _N, _D)
    pad = _EHOST - _E
    col = jnp.pad(edge_index[1], (0, pad))
    row = jnp.pad(edge_index[0], (0, pad))
    cols_all = jnp.concatenate([col, col + _N])
    adj_all = jnp.concatenate([jnp.pad(adj0_values, (0, pad)),
                               jnp.pad(adj1_values, (0, pad))])
    m_out, v_out = _sc_aggregate()(x_all, cols_all, row, adj_all)
    return (m_out, v_out)


# speedup vs baseline: 1.9385x; 1.0194x over previous
import functools

import jax
import jax.numpy as jnp
from jax import lax
from jax.experimental import pallas as pl
from jax.experimental.pallas import tpu as pltpu
from jax.experimental.pallas import tpu_sc as plsc

_N = 10000
_D = 128
_E = 320000
_RB = 1000
_CH = 112
_NSUB = 16
_CPT = 180
_EPAD = _NSUB * _CPT * _CH
_EHOST = _EPAD + 3 * _CH
_RPT = 624
_TAIL0 = _NSUB * _RPT
_TAILN = _N - _TAIL0
_UNROLL = 15


def _tc_body(mean_ref, var_ref, wm_ref, bm_ref, wv_ref, bv_ref, out_ref):
    dn = (((1,), (1,)), ((), ()))
    ml = lax.dot_general(mean_ref[...], wm_ref[...], dn,
                         preferred_element_type=jnp.float32) + bm_ref[...]
    vl = lax.dot_general(var_ref[...], wv_ref[...], dn,
                         preferred_element_type=jnp.float32) + bv_ref[...]
    me = jnp.where(ml > 0, ml, jnp.exp(ml) - 1.0)
    vr = jnp.maximum(vl, 0.0)
    att = jnp.exp(-vr)
    out_ref[0] = me * att
    out_ref[1] = vr * att * att


def _tc_transform(mean, var, w_mean, b_mean, w_var, b_var):
    nb = _N // _RB
    return pl.pallas_call(
        _tc_body,
        grid=(nb,),
        in_specs=[
            pl.BlockSpec((_RB, _D), lambda b: (b, 0)),
            pl.BlockSpec((_RB, _D), lambda b: (b, 0)),
            pl.BlockSpec((_D, _D), lambda b: (0, 0)),
            pl.BlockSpec((1, _D), lambda b: (0, 0)),
            pl.BlockSpec((_D, _D), lambda b: (0, 0)),
            pl.BlockSpec((1, _D), lambda b: (0, 0)),
        ],
        out_specs=pl.BlockSpec((2, _RB, _D), lambda b: (0, b, 0)),
        out_shape=jax.ShapeDtypeStruct((2, _N, _D), jnp.float32),
    )(mean, var, w_mean, b_mean.reshape(1, _D), w_var, b_var.reshape(1, _D))


def _sc_body(x_hbm, cols_hbm, rows_hbm, adj_hbm, m_out, v_out,
             col_r, row_r, w_r, buf0, buf1, buf2, acc, sems):
    cid = lax.axis_index("c")
    sid = lax.axis_index("s")
    bufs = (buf0, buf1, buf2)
    e00 = sid * _CPT * _CH

    def _stage_idx(c, slot):
        e0 = cid * _EHOST + e00 + c * _CH
        pltpu.async_copy(cols_hbm.at[pl.ds(e0, _CH)],
                         col_r.at[slot], sems[6 + slot])
        pltpu.async_copy(rows_hbm.at[pl.ds(e00 + c * _CH, _CH)],
                         row_r.at[slot], sems[6 + slot])
        pltpu.async_copy(adj_hbm.at[pl.ds(e0, _CH)],
                         w_r.at[slot], sems[6 + slot])

    def _wait_idx(slot):
        pltpu.make_async_copy(cols_hbm.at[pl.ds(0, _CH)],
                              col_r.at[slot], sems[6 + slot]).wait()
        pltpu.make_async_copy(rows_hbm.at[pl.ds(0, _CH)],
                              row_r.at[slot], sems[6 + slot]).wait()
        pltpu.make_async_copy(adj_hbm.at[pl.ds(0, _CH)],
                              w_r.at[slot], sems[6 + slot]).wait()

    def _gather(slot, b):
        pltpu.async_copy(x_hbm.at[col_r.at[slot]], bufs[b], sems[b])

    def _wait_gather(b):
        pltpu.make_async_copy(x_hbm.at[col_r.at[0]], bufs[b], sems[b]).wait()

    def _scatter(slot, b):
        pltpu.async_copy(bufs[b], acc.at[row_r.at[slot]], sems[3 + b],
                         add=True)

    def _wait_scatter(b):
        pltpu.make_async_copy(bufs[b], acc.at[row_r.at[0]],
                              sems[3 + b]).wait()

    def _scale(slot, b):
        buf = bufs[b]

        def _grp(g, carry):
            wv = w_r[slot, pl.ds(g * 16, 16)]
            for k in range(16):
                wbc = jnp.broadcast_to(wv[k], (16,))
                e = g * 16 + k
                for fb in range(8):
                    sl = pl.ds(fb * 16, 16)
                    buf[e, sl] = buf[e, sl] * wbc
            return carry
        lax.fori_loop(0, _CH // 16, _grp, 0)

    for c in range(3):
        _stage_idx(c, c)

    def _zrow(i, carry):
        for b in range(8):
            buf0[i, pl.ds(b * 16, 16)] = jnp.zeros((16,), jnp.float32)
        return carry
    lax.fori_loop(0, _CH, _zrow, 0)
    r0 = sid * _RPT
    for k in range(5):
        pltpu.sync_copy(buf0, acc.at[pl.ds(r0 + k * _CH, _CH)])
    pltpu.sync_copy(buf0.at[pl.ds(0, _RPT - 5 * _CH)],
                    acc.at[pl.ds(r0 + 5 * _CH, _RPT - 5 * _CH)])

    @pl.when(sid == _NSUB - 1)
    def _():
        pltpu.sync_copy(buf0.at[pl.ds(0, _TAILN)],
                        acc.at[pl.ds(_TAIL0, _TAILN)])
    plsc.subcore_barrier()

    pltpu.async_copy(acc.at[pl.ds(0, _CH)], buf2, sems[3 + 2])

    _wait_idx(0)
    _wait_idx(1)
    _gather(0, 0)
    _gather(1, 1)

    def _body(it, carry):
        c_base = it * _UNROLL
        for u in range(_UNROLL):
            b = u % 3
            s = u % 5
            _wait_gather(b)
            _scale(s, b)
            _wait_scatter((u + 2) % 3)
            _wait_idx((u + 2) % 5)
            _gather((u + 2) % 5, (u + 2) % 3)
            _stage_idx(c_base + u + 3, (u + 3) % 5)
            _scatter(s, b)
        return carry
    lax.fori_loop(0, _CPT // _UNROLL, _body, 0)

    _wait_gather(0)
    _wait_gather(1)
    _wait_idx(182 % 5)
    _wait_scatter(179 % 3)
    plsc.subcore_barrier()

    @pl.when(cid == 0)
    def _():
        pltpu.sync_copy(acc.at[pl.ds(r0, _RPT)], m_out.at[pl.ds(r0, _RPT)])

        @pl.when(sid == _NSUB - 1)
        def _():
            pltpu.sync_copy(acc.at[pl.ds(_TAIL0, _TAILN)],
                            m_out.at[pl.ds(_TAIL0, _TAILN)])

    @pl.when(cid == 1)
    def _():
        pltpu.sync_copy(acc.at[pl.ds(r0, _RPT)], v_out.at[pl.ds(r0, _RPT)])

        @pl.when(sid == _NSUB - 1)
        def _():
            pltpu.sync_copy(acc.at[pl.ds(_TAIL0, _TAILN)],
                            v_out.at[pl.ds(_TAIL0, _TAILN)])


@functools.cache
def _sc_aggregate():
    return functools.partial(
        pl.kernel,
        out_type=[jax.ShapeDtypeStruct((_N, _D), jnp.float32),
                  jax.ShapeDtypeStruct((_N, _D), jnp.float32)],
        mesh=plsc.VectorSubcoreMesh(core_axis_name="c", subcore_axis_name="s",
                                    num_cores=2, num_subcores=_NSUB),
        scratch_types=[
            pltpu.VMEM((5, _CH), jnp.int32),
            pltpu.VMEM((5, _CH), jnp.int32),
            pltpu.VMEM((5, _CH), jnp.float32),
            pltpu.VMEM((_CH, _D), jnp.float32),
            pltpu.VMEM((_CH, _D), jnp.float32),
            pltpu.VMEM((_CH, _D), jnp.float32),
            pltpu.VMEM_SHARED((_N, _D), jnp.float32),
            [pltpu.SemaphoreType.DMA] * 11,
        ],
    )(_sc_body)


def kernel(mean, var, edge_index, adj0_values, adj1_values,
           W_mean, b_mean, W_var, b_var):
    x_all = _tc_transform(mean, var, W_mean, b_mean, W_var, b_var)
    x_all = x_all.reshape(2 * _N, _D)
    pad = _EHOST - _E
    col = jnp.pad(edge_index[1], (0, pad))
    row = jnp.pad(edge_index[0], (0, pad))
    cols_all = jnp.concatenate([col, col + _N])
    adj_all = jnp.concatenate([jnp.pad(adj0_values, (0, pad)),
                               jnp.pad(adj1_values, (0, pad))])
    m_out, v_out = _sc_aggregate()(x_all, cols_all, row, adj_all)
    return (m_out, v_out)
